# baseline (speedup 1.0000x reference)
import jax
import jax.numpy as jnp
from jax import lax
from jax.experimental import pallas as pl
from jax.experimental.pallas import tpu as pltpu
from jax.experimental.pallas import tpu_sc as plsc

N = 10000
D = 128
NPAD = 10240
E = 320000
CH = 128
SLABS = 16
CPS = 160
EPAD = SLABS * CPS * CH
NC = 2
NS = 16
ROWS_PER_TILE = NPAD // (NC * NS)
ROWS_PER_SC_TILE = NPAD // NS

import functools


@functools.cache
def _mesh():
  return plsc.VectorSubcoreMesh(core_axis_name="c", subcore_axis_name="s",
                                num_cores=NC, num_subcores=NS)


def _zero_vmem_2d(ref, nrows):
  z = jnp.zeros((16,), ref.dtype)

  def body(r, _):
    for cc in range(ref.shape[1] // 16):
      ref[r, pl.ds(cc * 16, 16)] = z
    return 0

  lax.fori_loop(0, nrows, body, 0)


def _deg_kernel(dst_hbm, degb_hbm, dslab_v, ones_v, degv, out_v, zb,
                shist):
  c = lax.axis_index("c")
  s = lax.axis_index("s")
  wid = c * NS + s

  def zb_body(i, _):
    zb[pl.ds(i * 16, 16)] = jnp.zeros((16,), jnp.int32)
    return 0
  lax.fori_loop(0, ROWS_PER_SC_TILE // 16, zb_body, 0)
  pltpu.sync_copy(zb, shist.at[pl.ds(s * ROWS_PER_SC_TILE, ROWS_PER_SC_TILE)])
  plsc.subcore_barrier()

  rows_per_tile_hist = (EPAD // CH) // NS
  pltpu.sync_copy(dst_hbm.at[pl.ds(s * rows_per_tile_hist, rows_per_tile_hist)],
                  dslab_v)

  def ones_body(i, _):
    ones_v[pl.ds(i * 16, 16)] = jnp.ones((16,), jnp.int32)
    return 0
  lax.fori_loop(0, CH // 16, ones_body, 0)

  def hist_body(k, _):
    pltpu.sync_copy(ones_v, shist.at[dslab_v.at[k]], add=True)
    return 0
  lax.fori_loop(0, rows_per_tile_hist, hist_body, 0)
  plsc.subcore_barrier()

  r0 = wid * ROWS_PER_TILE
  pltpu.sync_copy(shist.at[pl.ds(r0, ROWS_PER_TILE)], degv)

  def splat_body(i, _):
    v = (degv[pl.ds(i * 16, 16)] + 1).astype(jnp.float32)
    for j in range(16):
      row = i * 16 + j
      vv = jnp.full((16,), v[j], jnp.float32)
      for cc in range(8):
        out_v[row, pl.ds(cc * 16, 16)] = vv
    return 0
  lax.fori_loop(0, ROWS_PER_TILE // 16, splat_body, 0)

  pltpu.sync_copy(out_v, degb_hbm.at[pl.ds(r0, ROWS_PER_TILE)])


@functools.cache
def _deg_call():
  return pl.kernel(
    _deg_kernel,
    out_type=jax.ShapeDtypeStruct((NPAD, D), jnp.float32),
    mesh=_mesh(),
    scratch_types=[
        pltpu.VMEM(((EPAD // CH) // NS, CH), jnp.int32),
        pltpu.VMEM((CH,), jnp.int32),
        pltpu.VMEM((ROWS_PER_TILE,), jnp.int32),
        pltpu.VMEM((ROWS_PER_TILE, D), jnp.float32),
        pltpu.VMEM((ROWS_PER_SC_TILE,), jnp.int32),
        pltpu.VMEM_SHARED((NPAD,), jnp.int32),
    ],
  )


HD = D // 2


NBUF = 6
HALVES = 2
CPH = CPS // HALVES


def _agg_kernel(uh_hbm, src_hbm, dst_hbm, parts_hbm, srcs, dsts,
                bufs, buff, sems, acc):
  c = lax.axis_index("c")
  s = lax.axis_index("s")

  _zero_vmem_2d(buff, CH)
  for k in range(ROWS_PER_SC_TILE // CH):
    pltpu.sync_copy(buff, acc.at[pl.ds(s * ROWS_PER_SC_TILE + k * CH, CH)])
  plsc.subcore_barrier()

  def start(g, buf, sem):
    pltpu.async_copy(uh_hbm.at[srcs.at[g]], buf, sem)

  def wait(g, buf, sem):
    pltpu.make_async_copy(uh_hbm.at[srcs.at[g]], buf, sem).wait()

  def convert(bh):
    RU = 4

    def cblk(i, _):
      for rr in range(RU):
        r = i * RU + rr
        for gg in range(HD // 32):
          v = bh[r, pl.ds(gg * 16, 16)]
          lo = plsc.bitcast(v << 16, jnp.float32)
          hi = plsc.bitcast(v & jnp.int32(-65536), jnp.float32)
          buff[r, pl.ds(gg * 32, 16)] = lo
          buff[r, pl.ds(gg * 32 + 16, 16)] = hi
      return 0
    lax.fori_loop(0, CH // RU, cblk, 0)

  for h in range(HALVES):
    pltpu.sync_copy(src_hbm.at[(c * SLABS + s) * HALVES + h], srcs)
    pltpu.sync_copy(dst_hbm.at[s * HALVES + h], dsts)

    for g0 in range(NBUF - 1):
      start(g0, bufs[g0], sems[g0])

    def body(g, _):
      nxt = g + NBUF - 1
      for k in range(NBUF):
        @pl.when(jnp.logical_and(nxt < CPH, nxt % NBUF == k))
        def _(k=k, nxt=nxt):
          start(nxt, bufs[k], sems[k])
      for k in range(NBUF):
        @pl.when(g % NBUF == k)
        def _(k=k, g=g):
          wait(g, bufs[k], sems[k])
          convert(bufs[k])
          pltpu.sync_copy(buff, acc.at[dsts.at[g]], add=True)
      return 0

    lax.fori_loop(0, CPH, body, 0)

  plsc.subcore_barrier()

  r0 = s * ROWS_PER_SC_TILE
  pltpu.sync_copy(acc.at[pl.ds(r0, ROWS_PER_SC_TILE)],
                  parts_hbm.at[pl.ds(c * NPAD + r0, ROWS_PER_SC_TILE)])


@functools.cache
def _agg_call():
  return pl.kernel(
    _agg_kernel,
    out_type=jax.ShapeDtypeStruct((NC * NPAD, HD), jnp.float32),
    mesh=_mesh(),
    scratch_types=[
        pltpu.VMEM((CPH, CH), jnp.int32),
        pltpu.VMEM((CPH, CH), jnp.int32),
        [pltpu.VMEM((CH, HD // 2), jnp.int32)] * NBUF,
        pltpu.VMEM((CH, HD), jnp.float32),
        [pltpu.SemaphoreType.DMA] * NBUF,
        pltpu.VMEM_SHARED((NPAD, HD), jnp.float32),
    ],
    compiler_params=pltpu.CompilerParams(use_tc_tiling_on_sc=False,
                                         needs_layout_passes=False),
  )


BLK = 512
GRID = NPAD // BLK


def _mm_scale_kernel(x_ref, w_ref, wp_ref, degb_ref, o_ref, oh_ref):
  dinv = lax.rsqrt(degb_ref[...])
  x = x_ref[...]
  h = jnp.dot(x, w_ref[...], preferred_element_type=jnp.float32)
  o_ref[...] = h * dinv
  hp = jnp.dot(x, wp_ref[...], preferred_element_type=jnp.float32)
  oh_ref[...] = (hp * dinv).astype(jnp.bfloat16)


def _mid_kernel(pl_ref, pr_ref, u_ref, degb_ref, b_ref, w_ref, wp_ref,
                o_ref, oh_ref):
  dinv = lax.rsqrt(degb_ref[...])
  u = u_ref[...]
  b = b_ref[...]
  hl = dinv[:, :HD] * (pl_ref[...] + u[:, :HD]) + b[:, :HD]
  hr = dinv[:, HD:] * (pr_ref[...] + u[:, HD:]) + b[:, HD:]
  hl = jnp.maximum(hl, 0.0)
  hr = jnp.maximum(hr, 0.0)
  h = jnp.dot(hl, w_ref[:HD, :], preferred_element_type=jnp.float32) \
      + jnp.dot(hr, w_ref[HD:, :], preferred_element_type=jnp.float32)
  o_ref[...] = h * dinv
  hp = jnp.dot(hl, wp_ref[:HD, :], preferred_element_type=jnp.float32) \
      + jnp.dot(hr, wp_ref[HD:, :], preferred_element_type=jnp.float32)
  oh_ref[...] = (hp * dinv).astype(jnp.bfloat16)


def _final_kernel(pl_ref, pr_ref, u_ref, degb_ref, b_ref, o_ref):
  dinv = lax.rsqrt(degb_ref[...])
  u = u_ref[...]
  b = b_ref[...]
  o_ref[:, :HD] = dinv[:, :HD] * (pl_ref[...] + u[:, :HD]) + b[:, :HD]
  o_ref[:, HD:] = dinv[:, HD:] * (pr_ref[...] + u[:, HD:]) + b[:, HD:]


_row_spec = pl.BlockSpec((BLK, D), lambda i: (i, 0))
_half_spec = pl.BlockSpec((BLK, HD), lambda i: (i, 0))
_full_spec = pl.BlockSpec((D, D), lambda i: (0, 0))
_b_spec = pl.BlockSpec((1, D), lambda i: (0, 0))
_out_sds = jax.ShapeDtypeStruct((NPAD, D), jnp.float32)

_outh_sds = jax.ShapeDtypeStruct((NPAD, D), jnp.bfloat16)

_mm_scale = pl.pallas_call(
    _mm_scale_kernel,
    grid=(GRID,),
    in_specs=[_row_spec, _full_spec, _full_spec, _row_spec],
    out_specs=[_row_spec, _row_spec],
    out_shape=[_out_sds, _outh_sds],
)

_mid = pl.pallas_call(
    _mid_kernel,
    grid=(GRID,),
    in_specs=[_half_spec, _half_spec, _row_spec, _row_spec, _b_spec,
              _full_spec, _full_spec],
    out_specs=[_row_spec, _row_spec],
    out_shape=[_out_sds, _outh_sds],
)

_final = pl.pallas_call(
    _final_kernel,
    grid=(GRID,),
    in_specs=[_half_spec, _half_spec, _row_spec, _row_spec, _b_spec],
    out_specs=_row_spec,
    out_shape=_out_sds,
)


@jax.jit
def _run(x, edge_index, W1, b1, W2, b2):
  src = edge_index[0]
  dst = edge_index[1]
  pad = EPAD - E
  fill = jnp.full((pad,), NPAD - 1, jnp.int32)
  src_pc = jnp.concatenate([src, fill])
  src2 = jnp.stack([src_pc * 2, src_pc * 2 + 1])
  src_p = src2.reshape(NC * SLABS * HALVES, CPH, CH)
  dst_pc = jnp.concatenate([dst, fill])
  dst_flat = dst_pc.reshape(EPAD // CH, CH)
  dst_p = dst_pc.reshape(SLABS * HALVES, CPH, CH)

  x_p = jnp.zeros((NPAD, D), x.dtype).at[:N].set(x)
  b1r = b1.reshape(1, D)
  b2r = b2.reshape(1, D)

  perm = []
  for g in range(D // 32):
    for j in range(16):
      perm += [g * 32 + j, g * 32 + 16 + j]
  perm = jnp.array(perm, jnp.int32)
  W1p = W1[:, perm]
  W2p = W2[:, perm]

  def as_i32(uh):
    return lax.bitcast_convert_type(
        uh.reshape(2 * NPAD, HD // 2, 2), jnp.int32)

  degb = _deg_call()(dst_flat)
  u1, uh1 = _mm_scale(x_p, W1, W1p, degb)
  parts1 = _agg_call()(as_i32(uh1), src_p, dst_p)
  u2, uh2 = _mid(parts1[:NPAD], parts1[NPAD:], u1, degb, b1r, W2, W2p)
  parts2 = _agg_call()(as_i32(uh2), src_p, dst_p)
  out = _final(parts2[:NPAD], parts2[NPAD:], u2, degb, b2r)
  return out[:N]


def kernel(x, edge_index, orbits, W1, b1, W2, b2):
  del orbits
  return _run(x, edge_index, W1, b1, W2, b2)


# speedup vs baseline: 2.7010x; 2.7010x over previous
import jax
import jax.numpy as jnp
from jax import lax
from jax.experimental import pallas as pl
from jax.experimental.pallas import tpu as pltpu
from jax.experimental.pallas import tpu_sc as plsc

N = 10000
D = 128
NPAD = 10240
E = 320000
CH = 128
SLABS = 16
CPS = 160
EPAD = SLABS * CPS * CH
NC = 2
NS = 16
ROWS_PER_TILE = NPAD // (NC * NS)
ROWS_PER_SC_TILE = NPAD // NS

import functools


@functools.cache
def _mesh():
  return plsc.VectorSubcoreMesh(core_axis_name="c", subcore_axis_name="s",
                                num_cores=NC, num_subcores=NS)


def _zero_vmem_2d(ref, nrows):
  z = jnp.zeros((16,), ref.dtype)

  def body(r, _):
    for cc in range(ref.shape[1] // 16):
      ref[r, pl.ds(cc * 16, 16)] = z
    return 0

  lax.fori_loop(0, nrows, body, 0)


def _deg_kernel(dst_hbm, degb_hbm, dslab_v, ones_v, degv, out_v, zb,
                shist):
  c = lax.axis_index("c")
  s = lax.axis_index("s")
  wid = c * NS + s

  def zb_body(i, _):
    zb[pl.ds(i * 16, 16)] = jnp.zeros((16,), jnp.int32)
    return 0
  lax.fori_loop(0, ROWS_PER_SC_TILE // 16, zb_body, 0)
  pltpu.sync_copy(zb, shist.at[pl.ds(s * ROWS_PER_SC_TILE, ROWS_PER_SC_TILE)])
  plsc.subcore_barrier()

  rows_per_tile_hist = (EPAD // CH) // NS
  pltpu.sync_copy(dst_hbm.at[pl.ds(s * rows_per_tile_hist, rows_per_tile_hist)],
                  dslab_v)

  def ones_body(i, _):
    ones_v[pl.ds(i * 16, 16)] = jnp.ones((16,), jnp.int32)
    return 0
  lax.fori_loop(0, CH // 16, ones_body, 0)

  def hist_body(k, _):
    pltpu.sync_copy(ones_v, shist.at[dslab_v.at[k]], add=True)
    return 0
  lax.fori_loop(0, rows_per_tile_hist, hist_body, 0)
  plsc.subcore_barrier()

  r0 = wid * ROWS_PER_TILE
  pltpu.sync_copy(shist.at[pl.ds(r0, ROWS_PER_TILE)], degv)

  def splat_body(i, _):
    v = (degv[pl.ds(i * 16, 16)] + 1).astype(jnp.float32)
    for j in range(16):
      row = i * 16 + j
      vv = jnp.full((16,), v[j], jnp.float32)
      for cc in range(8):
        out_v[row, pl.ds(cc * 16, 16)] = vv
    return 0
  lax.fori_loop(0, ROWS_PER_TILE // 16, splat_body, 0)

  pltpu.sync_copy(out_v, degb_hbm.at[pl.ds(r0, ROWS_PER_TILE)])


@functools.cache
def _deg_call():
  return pl.kernel(
    _deg_kernel,
    out_type=jax.ShapeDtypeStruct((NPAD, D), jnp.float32),
    mesh=_mesh(),
    scratch_types=[
        pltpu.VMEM(((EPAD // CH) // NS, CH), jnp.int32),
        pltpu.VMEM((CH,), jnp.int32),
        pltpu.VMEM((ROWS_PER_TILE,), jnp.int32),
        pltpu.VMEM((ROWS_PER_TILE, D), jnp.float32),
        pltpu.VMEM((ROWS_PER_SC_TILE,), jnp.int32),
        pltpu.VMEM_SHARED((NPAD,), jnp.int32),
    ],
  )


HD = D // 2


NBUF = 6
HALVES = 2
CPH = CPS // HALVES


def _agg_kernel(uh_hbm, src_hbm, dst_hbm, parts_hbm, srcs, dsts,
                bufs, buff, sems, acc):
  c = lax.axis_index("c")
  s = lax.axis_index("s")

  _zero_vmem_2d(buff, CH)
  for k in range(ROWS_PER_SC_TILE // CH):
    pltpu.sync_copy(buff, acc.at[pl.ds(s * ROWS_PER_SC_TILE + k * CH, CH)])
  plsc.subcore_barrier()

  def start(g, buf, sem):
    pltpu.async_copy(uh_hbm.at[srcs.at[g]], buf, sem)

  def wait(g, buf, sem):
    pltpu.make_async_copy(uh_hbm.at[srcs.at[g]], buf, sem).wait()

  def convert(bh):
    RU = 4

    def cblk(i, _):
      for rr in range(RU):
        r = i * RU + rr
        for gg in range(HD // 32):
          v = plsc.bitcast(bh[r, pl.ds(gg * 32, 32)], jnp.int32)
          lo = plsc.bitcast(v << 16, jnp.float32)
          hi = plsc.bitcast(v & jnp.int32(-65536), jnp.float32)
          buff[r, pl.ds(gg * 32, 16)] = lo
          buff[r, pl.ds(gg * 32 + 16, 16)] = hi
      return 0
    lax.fori_loop(0, CH // RU, cblk, 0)

  for h in range(HALVES):
    pltpu.sync_copy(src_hbm.at[(c * SLABS + s) * HALVES + h], srcs)
    pltpu.sync_copy(dst_hbm.at[s * HALVES + h], dsts)

    for g0 in range(NBUF - 1):
      start(g0, bufs[g0], sems[g0])

    def body(g, _):
      nxt = g + NBUF - 1
      for k in range(NBUF):
        @pl.when(jnp.logical_and(nxt < CPH, nxt % NBUF == k))
        def _(k=k, nxt=nxt):
          start(nxt, bufs[k], sems[k])
      for k in range(NBUF):
        @pl.when(g % NBUF == k)
        def _(k=k, g=g):
          wait(g, bufs[k], sems[k])
          convert(bufs[k])
          pltpu.sync_copy(buff, acc.at[dsts.at[g]], add=True)
      return 0

    lax.fori_loop(0, CPH, body, 0)

  plsc.subcore_barrier()

  r0 = s * ROWS_PER_SC_TILE
  pltpu.sync_copy(acc.at[pl.ds(r0, ROWS_PER_SC_TILE)],
                  parts_hbm.at[pl.ds(c * NPAD + r0, ROWS_PER_SC_TILE)])


@functools.cache
def _agg_call():
  return pl.kernel(
    _agg_kernel,
    out_type=jax.ShapeDtypeStruct((NC * NPAD, HD), jnp.float32),
    mesh=_mesh(),
    scratch_types=[
        pltpu.VMEM((CPH, CH), jnp.int32),
        pltpu.VMEM((CPH, CH), jnp.int32),
        [pltpu.VMEM((CH, HD), jnp.bfloat16)] * NBUF,
        pltpu.VMEM((CH, HD), jnp.float32),
        [pltpu.SemaphoreType.DMA] * NBUF,
        pltpu.VMEM_SHARED((NPAD, HD), jnp.float32),
    ],
    compiler_params=pltpu.CompilerParams(use_tc_tiling_on_sc=False,
                                         needs_layout_passes=False),
  )


BLK = 512
GRID = NPAD // BLK


def _mm_scale_kernel(x_ref, w_ref, wp_ref, degb_ref, o_ref, oh_ref):
  dinv = lax.rsqrt(degb_ref[...])
  x = x_ref[...]
  h = jnp.dot(x, w_ref[...], preferred_element_type=jnp.float32)
  o_ref[...] = h * dinv
  hp = jnp.dot(x, wp_ref[...], preferred_element_type=jnp.float32)
  oh_ref[...] = (hp * dinv).astype(jnp.bfloat16)


def _mid_kernel(pl_ref, pr_ref, u_ref, degb_ref, b_ref, w_ref, wp_ref,
                o_ref, oh_ref):
  dinv = lax.rsqrt(degb_ref[...])
  u = u_ref[...]
  b = b_ref[...]
  hl = dinv[:, :HD] * (pl_ref[...] + u[:, :HD]) + b[:, :HD]
  hr = dinv[:, HD:] * (pr_ref[...] + u[:, HD:]) + b[:, HD:]
  hl = jnp.maximum(hl, 0.0)
  hr = jnp.maximum(hr, 0.0)
  h = jnp.dot(hl, w_ref[:HD, :], preferred_element_type=jnp.float32) \
      + jnp.dot(hr, w_ref[HD:, :], preferred_element_type=jnp.float32)
  o_ref[...] = h * dinv
  hp = jnp.dot(hl, wp_ref[:HD, :], preferred_element_type=jnp.float32) \
      + jnp.dot(hr, wp_ref[HD:, :], preferred_element_type=jnp.float32)
  oh_ref[...] = (hp * dinv).astype(jnp.bfloat16)


def _final_kernel(pl_ref, pr_ref, u_ref, degb_ref, b_ref, o_ref):
  dinv = lax.rsqrt(degb_ref[...])
  u = u_ref[...]
  b = b_ref[...]
  o_ref[:, :HD] = dinv[:, :HD] * (pl_ref[...] + u[:, :HD]) + b[:, :HD]
  o_ref[:, HD:] = dinv[:, HD:] * (pr_ref[...] + u[:, HD:]) + b[:, HD:]


_row_spec = pl.BlockSpec((BLK, D), lambda i: (i, 0))
_half_spec = pl.BlockSpec((BLK, HD), lambda i: (i, 0))
_full_spec = pl.BlockSpec((D, D), lambda i: (0, 0))
_b_spec = pl.BlockSpec((1, D), lambda i: (0, 0))
_out_sds = jax.ShapeDtypeStruct((NPAD, D), jnp.float32)

_outh_sds = jax.ShapeDtypeStruct((NPAD, D), jnp.bfloat16)

_mm_scale = pl.pallas_call(
    _mm_scale_kernel,
    grid=(GRID,),
    in_specs=[_row_spec, _full_spec, _full_spec, _row_spec],
    out_specs=[_row_spec, _row_spec],
    out_shape=[_out_sds, _outh_sds],
)

_mid = pl.pallas_call(
    _mid_kernel,
    grid=(GRID,),
    in_specs=[_half_spec, _half_spec, _row_spec, _row_spec, _b_spec,
              _full_spec, _full_spec],
    out_specs=[_row_spec, _row_spec],
    out_shape=[_out_sds, _outh_sds],
)

_final = pl.pallas_call(
    _final_kernel,
    grid=(GRID,),
    in_specs=[_half_spec, _half_spec, _row_spec, _row_spec, _b_spec],
    out_specs=_row_spec,
    out_shape=_out_sds,
)


@jax.jit
def _run(x, edge_index, W1, b1, W2, b2):
  src = edge_index[0]
  dst = edge_index[1]
  pad = EPAD - E
  fill = jnp.full((pad,), NPAD - 1, jnp.int32)
  src_pc = jnp.concatenate([src, fill])
  src2 = jnp.stack([src_pc * 2, src_pc * 2 + 1])
  src_p = src2.reshape(NC * SLABS * HALVES, CPH, CH)
  dst_pc = jnp.concatenate([dst, fill])
  dst_flat = dst_pc.reshape(EPAD // CH, CH)
  dst_p = dst_pc.reshape(SLABS * HALVES, CPH, CH)

  x_p = jnp.zeros((NPAD, D), x.dtype).at[:N].set(x)
  b1r = b1.reshape(1, D)
  b2r = b2.reshape(1, D)

  perm = []
  for g in range(D // 32):
    for j in range(16):
      perm += [g * 32 + j, g * 32 + 16 + j]
  perm = jnp.array(perm, jnp.int32)
  W1p = W1[:, perm]
  W2p = W2[:, perm]

  degb = _deg_call()(dst_flat)
  u1, uh1 = _mm_scale(x_p, W1, W1p, degb)
  parts1 = _agg_call()(uh1.reshape(2 * NPAD, HD), src_p, dst_p)
  u2, uh2 = _mid(parts1[:NPAD], parts1[NPAD:], u1, degb, b1r, W2, W2p)
  parts2 = _agg_call()(uh2.reshape(2 * NPAD, HD), src_p, dst_p)
  out = _final(parts2[:NPAD], parts2[NPAD:], u2, degb, b2r)
  return out[:N]


def kernel(x, edge_index, orbits, W1, b1, W2, b2):
  del orbits
  return _run(x, edge_index, W1, b1, W2, b2)


# speedup vs baseline: 4.0094x; 1.4844x over previous
import jax
import jax.numpy as jnp
from jax import lax
from jax.experimental import pallas as pl
from jax.experimental.pallas import tpu as pltpu
from jax.experimental.pallas import tpu_sc as plsc

N = 10000
D = 128
NPAD = 10240
E = 320000
CH = 128
SLABS = 16
CPS = 160
EPAD = SLABS * CPS * CH
NC = 2
NS = 16
ROWS_PER_TILE = NPAD // (NC * NS)
ROWS_PER_SC_TILE = NPAD // NS

import functools


@functools.cache
def _mesh():
  return plsc.VectorSubcoreMesh(core_axis_name="c", subcore_axis_name="s",
                                num_cores=NC, num_subcores=NS)


def _zero_vmem_2d(ref, nrows):
  w = 32 if ref.dtype.itemsize == 2 else 16
  z = jnp.zeros((w,), ref.dtype)

  def body(r, _):
    for cc in range(ref.shape[1] // w):
      ref[r, pl.ds(cc * w, w)] = z
    return 0

  lax.fori_loop(0, nrows, body, 0)


def _deg_kernel(dst_hbm, degb_hbm, dslab_v, ones_v, degv, out_v, zb,
                shist):
  c = lax.axis_index("c")
  s = lax.axis_index("s")
  wid = c * NS + s

  def zb_body(i, _):
    zb[pl.ds(i * 16, 16)] = jnp.zeros((16,), jnp.int32)
    return 0
  lax.fori_loop(0, ROWS_PER_SC_TILE // 16, zb_body, 0)
  pltpu.sync_copy(zb, shist.at[pl.ds(s * ROWS_PER_SC_TILE, ROWS_PER_SC_TILE)])
  plsc.subcore_barrier()

  rows_per_tile_hist = (EPAD // CH) // NS
  pltpu.sync_copy(dst_hbm.at[pl.ds(s * rows_per_tile_hist, rows_per_tile_hist)],
                  dslab_v)

  def ones_body(i, _):
    ones_v[pl.ds(i * 16, 16)] = jnp.ones((16,), jnp.int32)
    return 0
  lax.fori_loop(0, CH // 16, ones_body, 0)

  def hist_body(k, _):
    pltpu.sync_copy(ones_v, shist.at[dslab_v.at[k]], add=True)
    return 0
  lax.fori_loop(0, rows_per_tile_hist, hist_body, 0)
  plsc.subcore_barrier()

  r0 = wid * ROWS_PER_TILE
  pltpu.sync_copy(shist.at[pl.ds(r0, ROWS_PER_TILE)], degv)

  def splat_body(i, _):
    v = (degv[pl.ds(i * 16, 16)] + 1).astype(jnp.float32)
    for j in range(16):
      row = i * 16 + j
      vv = jnp.full((16,), v[j], jnp.float32)
      for cc in range(8):
        out_v[row, pl.ds(cc * 16, 16)] = vv
    return 0
  lax.fori_loop(0, ROWS_PER_TILE // 16, splat_body, 0)

  pltpu.sync_copy(out_v, degb_hbm.at[pl.ds(r0, ROWS_PER_TILE)])


@functools.cache
def _deg_call():
  return pl.kernel(
    _deg_kernel,
    out_type=jax.ShapeDtypeStruct((NPAD, D), jnp.float32),
    mesh=_mesh(),
    scratch_types=[
        pltpu.VMEM(((EPAD // CH) // NS, CH), jnp.int32),
        pltpu.VMEM((CH,), jnp.int32),
        pltpu.VMEM((ROWS_PER_TILE,), jnp.int32),
        pltpu.VMEM((ROWS_PER_TILE, D), jnp.float32),
        pltpu.VMEM((ROWS_PER_SC_TILE,), jnp.int32),
        pltpu.VMEM_SHARED((NPAD,), jnp.int32),
    ],
  )


HD = D // 2


NBUF = 6
HALVES = 2
CPH = CPS // HALVES


def _agg_kernel(uh_hbm, src_hbm, dst_hbm, parts_hbm, srcs, dsts,
                bufs, sems, acc):
  c = lax.axis_index("c")
  s = lax.axis_index("s")

  _zero_vmem_2d(bufs[0], CH)
  for k in range(ROWS_PER_SC_TILE // CH):
    pltpu.sync_copy(bufs[0], acc.at[pl.ds(s * ROWS_PER_SC_TILE + k * CH, CH)])
  plsc.subcore_barrier()

  def start(g, buf, sem):
    pltpu.async_copy(uh_hbm.at[srcs.at[g]], buf, sem)

  def wait(g, buf, sem):
    pltpu.make_async_copy(uh_hbm.at[srcs.at[g]], buf, sem).wait()

  for h in range(HALVES):
    pltpu.sync_copy(src_hbm.at[(c * SLABS + s) * HALVES + h], srcs)
    pltpu.sync_copy(dst_hbm.at[s * HALVES + h], dsts)

    for g0 in range(NBUF - 1):
      start(g0, bufs[g0], sems[g0])

    def body(g, _):
      nxt = g + NBUF - 1
      for k in range(NBUF):
        @pl.when(jnp.logical_and(nxt < CPH, nxt % NBUF == k))
        def _(k=k, nxt=nxt):
          start(nxt, bufs[k], sems[k])
      for k in range(NBUF):
        @pl.when(g % NBUF == k)
        def _(k=k, g=g):
          wait(g, bufs[k], sems[k])
          pltpu.sync_copy(bufs[k], acc.at[dsts.at[g]], add=True)
      return 0

    lax.fori_loop(0, CPH, body, 0)

  plsc.subcore_barrier()

  r0 = s * ROWS_PER_SC_TILE
  pltpu.sync_copy(acc.at[pl.ds(r0, ROWS_PER_SC_TILE)],
                  parts_hbm.at[pl.ds(c * NPAD + r0, ROWS_PER_SC_TILE)])


@functools.cache
def _agg_call():
  return pl.kernel(
    _agg_kernel,
    out_type=jax.ShapeDtypeStruct((NC * NPAD, HD), jnp.bfloat16),
    mesh=_mesh(),
    scratch_types=[
        pltpu.VMEM((CPH, CH), jnp.int32),
        pltpu.VMEM((CPH, CH), jnp.int32),
        [pltpu.VMEM((CH, HD), jnp.bfloat16)] * NBUF,
        [pltpu.SemaphoreType.DMA] * NBUF,
        pltpu.VMEM_SHARED((NPAD, HD), jnp.bfloat16),
    ],
    compiler_params=pltpu.CompilerParams(use_tc_tiling_on_sc=False,
                                         needs_layout_passes=False),
  )


BLK = 512
GRID = NPAD // BLK


def _mm_scale_kernel(x_ref, w_ref, degb_ref, o_ref, oh_ref):
  dinv = lax.rsqrt(degb_ref[...])
  h = jnp.dot(x_ref[...], w_ref[...], preferred_element_type=jnp.float32)
  u = h * dinv
  o_ref[...] = u
  oh_ref[...] = u.astype(jnp.bfloat16)


def _mid_kernel(pl_ref, pr_ref, u_ref, degb_ref, b_ref, w_ref,
                o_ref, oh_ref):
  dinv = lax.rsqrt(degb_ref[...])
  u = u_ref[...]
  b = b_ref[...]
  p0 = pl_ref[...].astype(jnp.float32)
  p1 = pr_ref[...].astype(jnp.float32)
  hl = dinv[:, :HD] * (p0 + u[:, :HD]) + b[:, :HD]
  hr = dinv[:, HD:] * (p1 + u[:, HD:]) + b[:, HD:]
  hl = jnp.maximum(hl, 0.0)
  hr = jnp.maximum(hr, 0.0)
  h = jnp.dot(hl, w_ref[:HD, :], preferred_element_type=jnp.float32) \
      + jnp.dot(hr, w_ref[HD:, :], preferred_element_type=jnp.float32)
  u2 = h * dinv
  o_ref[...] = u2
  oh_ref[...] = u2.astype(jnp.bfloat16)


def _final_kernel(pl_ref, pr_ref, u_ref, degb_ref, b_ref, o_ref):
  dinv = lax.rsqrt(degb_ref[...])
  u = u_ref[...]
  b = b_ref[...]
  p0 = pl_ref[...].astype(jnp.float32)
  p1 = pr_ref[...].astype(jnp.float32)
  o_ref[:, :HD] = dinv[:, :HD] * (p0 + u[:, :HD]) + b[:, :HD]
  o_ref[:, HD:] = dinv[:, HD:] * (p1 + u[:, HD:]) + b[:, HD:]


_row_spec = pl.BlockSpec((BLK, D), lambda i: (i, 0))
_half_spec = pl.BlockSpec((BLK, HD), lambda i: (i, 0))
_full_spec = pl.BlockSpec((D, D), lambda i: (0, 0))
_b_spec = pl.BlockSpec((1, D), lambda i: (0, 0))
_out_sds = jax.ShapeDtypeStruct((NPAD, D), jnp.float32)

_outh_sds = jax.ShapeDtypeStruct((NPAD, D), jnp.bfloat16)

_mm_scale = pl.pallas_call(
    _mm_scale_kernel,
    grid=(GRID,),
    in_specs=[_row_spec, _full_spec, _row_spec],
    out_specs=[_row_spec, _row_spec],
    out_shape=[_out_sds, _outh_sds],
)

_mid = pl.pallas_call(
    _mid_kernel,
    grid=(GRID,),
    in_specs=[_half_spec, _half_spec, _row_spec, _row_spec, _b_spec,
              _full_spec],
    out_specs=[_row_spec, _row_spec],
    out_shape=[_out_sds, _outh_sds],
)

_final = pl.pallas_call(
    _final_kernel,
    grid=(GRID,),
    in_specs=[_half_spec, _half_spec, _row_spec, _row_spec, _b_spec],
    out_specs=_row_spec,
    out_shape=_out_sds,
)


@jax.jit
def _run(x, edge_index, W1, b1, W2, b2):
  src = edge_index[0]
  dst = edge_index[1]
  pad = EPAD - E
  fill = jnp.full((pad,), NPAD - 1, jnp.int32)
  src_pc = jnp.concatenate([src, fill])
  src2 = jnp.stack([src_pc * 2, src_pc * 2 + 1])
  src_p = src2.reshape(NC * SLABS * HALVES, CPH, CH)
  dst_pc = jnp.concatenate([dst, fill])
  dst_flat = dst_pc.reshape(EPAD // CH, CH)
  dst_p = dst_pc.reshape(SLABS * HALVES, CPH, CH)

  x_p = jnp.zeros((NPAD, D), x.dtype).at[:N].set(x)
  b1r = b1.reshape(1, D)
  b2r = b2.reshape(1, D)

  degb = _deg_call()(dst_flat)
  u1, uh1 = _mm_scale(x_p, W1, degb)
  parts1 = _agg_call()(uh1.reshape(2 * NPAD, HD), src_p, dst_p)
  u2, uh2 = _mid(parts1[:NPAD], parts1[NPAD:], u1, degb, b1r, W2)
  parts2 = _agg_call()(uh2.reshape(2 * NPAD, HD), src_p, dst_p)
  out = _final(parts2[:NPAD], parts2[NPAD:], u2, degb, b2r)
  return out[:N]


def kernel(x, edge_index, orbits, W1, b1, W2, b2):
  del orbits
  return _run(x, edge_index, W1, b1, W2, b2)
